# baseline (speedup 1.0000x reference)
import functools
import math

import jax
import jax.numpy as jnp
from jax import lax
from jax.experimental import pallas as pl
from jax.experimental.pallas import tpu as pltpu
from jax.experimental.pallas import tpu_sc as plsc

DMODEL = 64
SCALE = math.sqrt(DMODEL)
_PRE = 2.0
_POST = 4.0

_NC = 2
_NS = 16
_NW = _NC * _NS

_LANES = 128
_B0 = 4096
_B1 = 200
_CTILES = _B0 // _LANES
_NBLK = _B1 * _CTILES
_BPW = _NBLK // _NW
_IDX_PER_W = _BPW * _LANES
_NBUF = 4


def _make_lookup():
    mesh = plsc.VectorSubcoreMesh(core_axis_name="c", subcore_axis_name="s")

    @functools.partial(
        pl.kernel,
        out_type=jax.ShapeDtypeStruct((_B1, 8, _CTILES, 8 * _LANES),
                                      jnp.float32),
        mesh=mesh,
        scratch_types=(
            [pltpu.VMEM((_IDX_PER_W,), jnp.int32)]
            + [pltpu.VMEM((_LANES, DMODEL), jnp.float32)] * _NBUF
            + [pltpu.VMEM((8, 8 * _LANES), jnp.float32)] * _NBUF
            + [pltpu.SemaphoreType.DMA] * (2 * _NBUF)
        ),
        compiler_params=pltpu.CompilerParams(use_tc_tiling_on_sc=False,
                                             needs_layout_passes=False),
    )
    def lookup(idx_hbm, table_hbm, out_hbm, idx_v, *bufs):
        rows = bufs[:_NBUF]
        tbuf = bufs[_NBUF:2 * _NBUF]
        gsem = bufs[2 * _NBUF:3 * _NBUF]
        ssem = bufs[3 * _NBUF:]
        wid = lax.axis_index("s") * _NC + lax.axis_index("c")
        jbase = wid * _BPW
        pltpu.sync_copy(idx_hbm.at[pl.ds(jbase * _LANES, _IDX_PER_W)], idx_v)

        iotas = [lax.iota(jnp.int32, 16) + 16 * k for k in range(8)]

        def start_gather(t, b):
            pltpu.async_copy(
                table_hbm.at[idx_v.at[pl.ds(t * _LANES, _LANES)]],
                rows[b], gsem[b])

        def wait_gather(b):
            pltpu.make_async_copy(
                table_hbm.at[idx_v.at[pl.ds(0, _LANES)]],
                rows[b], gsem[b]).wait()

        def start_store(t, b):
            j = jbase + t
            b1 = j // _CTILES
            c = j % _CTILES
            pltpu.async_copy(tbuf[b], out_hbm.at[b1, :, c], ssem[b])

        def wait_store(b):
            pltpu.make_async_copy(tbuf[b], out_hbm.at[0, :, 0],
                                  ssem[b]).wait()

        def transpose_scale(b):
            rb, tb = rows[b], tbuf[b]

            def body_j(j, carry):
                dloc = (iotas[0] + j) & 15
                rloc = dloc >> 3
                for m in range(DMODEL // 16):
                    d_vec = dloc + 16 * m
                    r_idx = rloc + 2 * m
                    inner_base = ((d_vec & 7) << 7)
                    for k in range(8):
                        v = plsc.load_gather(rb, [iotas[k], d_vec])
                        plsc.store_scatter(tb, [r_idx, inner_base + iotas[k]],
                                           v * SCALE)
                return carry

            lax.fori_loop(0, 16, body_j, 0, unroll=False)

        def step(t, b, wait_st, prefetch):
            wait_gather(b)
            if wait_st:
                wait_store(b)
            transpose_scale(b)
            start_store(t, b)
            if prefetch:
                start_gather(t + _NBUF, b)

        for b in range(_NBUF):
            start_gather(b, b)
        for b in range(_NBUF):
            step(b, b, wait_st=False, prefetch=True)

        def round_body(i, carry):
            for b in range(_NBUF):
                step(_NBUF * i + b, b, wait_st=True, prefetch=True)
            return carry

        lax.fori_loop(1, _BPW // _NBUF - 1, round_body, 0, unroll=False)

        for b in range(_NBUF):
            step(_BPW - _NBUF + b, b, wait_st=True, prefetch=False)
        for b in range(_NBUF):
            wait_store(b)

    return lookup


def kernel(x, table):
    idx = x.T.reshape(-1).astype(jnp.int32)
    out4 = _make_lookup()(idx, table)
    out5 = out4.reshape(_B1, 8, _CTILES, 8, _LANES)
    return out5.transpose(2, 4, 0, 1, 3).reshape(_B0, _B1, DMODEL)


# speedup vs baseline: 1.0439x; 1.0439x over previous
import functools
import math

import jax
import jax.numpy as jnp
from jax import lax
from jax.experimental import pallas as pl
from jax.experimental.pallas import tpu as pltpu
from jax.experimental.pallas import tpu_sc as plsc

DMODEL = 64
SCALE = math.sqrt(DMODEL)

_NC = 2
_NS = 16
_NW = _NC * _NS

_LANES = 128
_B0 = 4096
_B1 = 200
_CTILES = _B0 // _LANES
_NBLK = _B1 * _CTILES
_BPW = _NBLK // _NW
_IDX_PER_W = _BPW * _LANES
_NBUF = 2


def _make_lookup():
    mesh = plsc.VectorSubcoreMesh(core_axis_name="c", subcore_axis_name="s")

    @functools.partial(
        pl.kernel,
        out_type=jax.ShapeDtypeStruct((_B1, 8, _CTILES, 8, _LANES),
                                      jnp.float32),
        mesh=mesh,
        scratch_types=(
            [pltpu.VMEM((_IDX_PER_W,), jnp.int32)]
            + [pltpu.VMEM((_LANES, DMODEL), jnp.float32)] * _NBUF
            + [pltpu.VMEM((8, 8, _LANES), jnp.float32)] * _NBUF
            + [pltpu.SemaphoreType.DMA] * (2 * _NBUF)
        ),
        compiler_params=pltpu.CompilerParams(use_tc_tiling_on_sc=False,
                                             needs_layout_passes=False),
    )
    def lookup(idx_hbm, table_hbm, out_hbm, idx_v, *bufs):
        rows = bufs[:_NBUF]
        tbuf = bufs[_NBUF:2 * _NBUF]
        gsem = bufs[2 * _NBUF:3 * _NBUF]
        ssem = bufs[3 * _NBUF:]
        wid = lax.axis_index("s") * _NC + lax.axis_index("c")
        jbase = wid * _BPW
        pltpu.sync_copy(idx_hbm.at[pl.ds(jbase * _LANES, _IDX_PER_W)], idx_v)

        iotas = [lax.iota(jnp.int32, 16) + 16 * k for k in range(8)]

        def start_gather(t, b):
            pltpu.async_copy(
                table_hbm.at[idx_v.at[pl.ds(t * _LANES, _LANES)]],
                rows[b], gsem[b])

        def wait_gather(b):
            pltpu.make_async_copy(
                table_hbm.at[idx_v.at[pl.ds(0, _LANES)]],
                rows[b], gsem[b]).wait()

        def start_store(t, b):
            j = jbase + t
            b1 = j // _CTILES
            c = j % _CTILES
            pltpu.async_copy(tbuf[b], out_hbm.at[b1, :, c], ssem[b])

        def wait_store(b):
            pltpu.make_async_copy(tbuf[b], out_hbm.at[0, :, 0],
                                  ssem[b]).wait()

        def transpose_scale(b):
            rb, tb = rows[b], tbuf[b]

            def body_j(j, carry):
                dloc = (iotas[0] + j) & 15
                s_idx = dloc & 7
                rloc = dloc >> 3
                for m in range(DMODEL // 16):
                    d_vec = dloc + 16 * m
                    r_idx = rloc + 2 * m
                    for k in range(8):
                        v = plsc.load_gather(rb, [iotas[k], d_vec])
                        plsc.store_scatter(tb, [r_idx, s_idx, iotas[k]],
                                           v * SCALE)
                return carry

            lax.fori_loop(0, 16, body_j, 0, unroll=False)

        def step(t, b, wait_st, prefetch):
            wait_gather(b)
            if wait_st:
                wait_store(b)
            transpose_scale(b)
            start_store(t, b)
            if prefetch:
                start_gather(t + _NBUF, b)

        for b in range(_NBUF):
            start_gather(b, b)
        for b in range(_NBUF):
            step(b, b, wait_st=False, prefetch=True)

        def round_body(i, carry):
            for b in range(_NBUF):
                step(_NBUF * i + b, b, wait_st=True, prefetch=True)
            return carry

        lax.fori_loop(1, _BPW // _NBUF - 1, round_body, 0, unroll=False)

        for b in range(_NBUF):
            step(_BPW - _NBUF + b, b, wait_st=True, prefetch=False)
        for b in range(_NBUF):
            wait_store(b)

    return lookup


def kernel(x, table):
    idx = x.T.reshape(-1).astype(jnp.int32)
    out5 = _make_lookup()(idx, table)
    return out5.transpose(2, 4, 0, 1, 3).reshape(_B0, _B1, DMODEL)
